# Initial kernel scaffold; baseline (speedup 1.0000x reference)
import functools
import math

import jax
import jax.numpy as jnp
from jax import lax
from jax.experimental import pallas as pl
from jax.experimental.pallas import tpu as pltpu
from jax.experimental.pallas import tpu_sc as plsc

N_CORES = 2
N_SUBCORES = 16
N_TILES = N_CORES * N_SUBCORES
LANES = 128


@functools.lru_cache(maxsize=None)
def _make_sc_agg(n, f, k):
    rows = n // N_SUBCORES
    mesh = plsc.VectorSubcoreMesh(core_axis_name="c", subcore_axis_name="s")

    @functools.partial(
        pl.kernel,
        out_type=jax.ShapeDtypeStruct((N_CORES, n, f), jnp.float32),
        mesh=mesh,
        scratch_types=[
            pltpu.VMEM((k, LANES), jnp.int32),
            pltpu.VMEM((k, LANES), jnp.int32),
            pltpu.VMEM((LANES, f), jnp.float32),
            pltpu.VMEM_SHARED((n + LANES, f), jnp.float32),
        ],
    )
    def agg(g_hbm, src_hbm, dst_hbm, z_hbm, out_hbm, src_v, dst_v, buf, acc):
        c = lax.axis_index("c")
        s = lax.axis_index("s")
        wid = s * N_CORES + c
        pltpu.sync_copy(z_hbm, acc.at[pl.ds(s * rows, rows)])
        pltpu.sync_copy(src_hbm.at[wid], src_v)
        pltpu.sync_copy(dst_hbm.at[wid], dst_v)
        plsc.subcore_barrier()

        @pl.loop(0, k)
        def _(j):
            pltpu.sync_copy(g_hbm.at[src_v.at[j]], buf)
            pltpu.sync_copy(buf, acc.at[dst_v.at[j]], add=True)

        plsc.subcore_barrier()
        pltpu.sync_copy(
            acc.at[pl.ds(s * rows, rows)],
            out_hbm.at[c].at[pl.ds(s * rows, rows)],
        )

    return agg


@functools.lru_cache(maxsize=None)
def _make_sc_degree(n, f, k):
    rows = n // N_SUBCORES
    mesh = plsc.VectorSubcoreMesh(core_axis_name="c", subcore_axis_name="s")

    @functools.partial(
        pl.kernel,
        out_type=jax.ShapeDtypeStruct((N_CORES, n, f), jnp.float32),
        mesh=mesh,
        scratch_types=[
            pltpu.VMEM((k, LANES), jnp.int32),
            pltpu.VMEM((LANES, f), jnp.float32),
            pltpu.VMEM_SHARED((n + LANES, f), jnp.float32),
        ],
    )
    def degk(dst_hbm, ones_hbm, z_hbm, out_hbm, dst_v, ones_v, acc):
        c = lax.axis_index("c")
        s = lax.axis_index("s")
        wid = s * N_CORES + c
        pltpu.sync_copy(z_hbm, acc.at[pl.ds(s * rows, rows)])
        pltpu.sync_copy(dst_hbm.at[wid], dst_v)
        pltpu.sync_copy(ones_hbm, ones_v)
        plsc.subcore_barrier()

        @pl.loop(0, k)
        def _(j):
            pltpu.sync_copy(ones_v, acc.at[dst_v.at[j]], add=True)

        plsc.subcore_barrier()
        pltpu.sync_copy(
            acc.at[pl.ds(s * rows, rows)],
            out_hbm.at[c].at[pl.ds(s * rows, rows)],
        )

    return degk


def _dot(a, b):
    return jax.lax.dot_general(
        a, b, (((1,), (0,)), ((), ())),
        precision=jax.lax.Precision.HIGHEST,
        preferred_element_type=jnp.float32,
    )


def _m1_body(pdeg_ref, x_ref, w_ref, dinv_ref, g_ref):
    deg = pdeg_ref[0, :, :1] + pdeg_ref[1, :, :1] + 1.0
    dinv = jax.lax.rsqrt(jnp.maximum(deg, 1e-12))
    dinv_ref[...] = dinv
    g_ref[...] = _dot(x_ref[...], w_ref[...]) * dinv


def _m2_body(p_ref, g_ref, dinv_ref, w_ref, b_ref, out_ref):
    dinv = dinv_ref[...]
    h = jnp.maximum(dinv * (p_ref[0] + p_ref[1] + g_ref[...]) + b_ref[...], 0.0)
    out_ref[...] = _dot(h, w_ref[...]) * dinv


def _m4_body(p_ref, g_ref, dinv_ref, b_ref, out_ref):
    t = dinv_ref[...] * (p_ref[0] + p_ref[1] + g_ref[...])
    logits = t[:, :2] + b_ref[...]
    m = jnp.maximum(logits[:, :1], logits[:, 1:2])
    e0 = jnp.exp(logits[:, :1] - m)
    e1 = jnp.exp(logits[:, 1:2] - m)
    lse = jnp.log(e0 + e1) + m
    out_ref[...] = logits - lse


def _tc(body, out_shapes, *args):
    return pl.pallas_call(body, out_shape=out_shapes)(*args)


def kernel(x, edge_index, W1, b1, W2, b2, W3, b3):
    n, _ = x.shape
    e = edge_index.shape[1]
    k = math.ceil(e / (N_TILES * LANES))
    ep = N_TILES * LANES * k

    src = edge_index[0].astype(jnp.int32)
    dst = edge_index[1].astype(jnp.int32)
    pad = ep - e
    src_p = jnp.concatenate([src, jnp.zeros((pad,), jnp.int32)]).reshape(
        N_TILES, k, LANES)
    dst_p = jnp.concatenate([dst, jnp.full((pad,), n, jnp.int32)]).reshape(
        N_TILES, k, LANES)

    f1 = W1.shape[1]
    f2 = W2.shape[1]
    fd = 16
    rows = n // N_SUBCORES
    z1 = jnp.zeros((rows, f1), jnp.float32)
    z2 = jnp.zeros((rows, f2), jnp.float32)
    zd = jnp.zeros((rows, fd), jnp.float32)
    ones = jnp.ones((LANES, fd), jnp.float32)
    W3p = jnp.pad(W3, ((0, 0), (0, fd - W3.shape[1])))

    pdeg = _make_sc_degree(n, fd, k)(dst_p, ones, zd)
    dinv, g1 = _tc(
        _m1_body,
        (jax.ShapeDtypeStruct((n, 1), jnp.float32),
         jax.ShapeDtypeStruct((n, f1), jnp.float32)),
        pdeg, x, W1)
    p1 = _make_sc_agg(n, f1, k)(g1, src_p, dst_p, z1)
    g2 = _tc(_m2_body, jax.ShapeDtypeStruct((n, f2), jnp.float32),
             p1, g1, dinv, W2, b1.reshape(1, -1))
    p2 = _make_sc_agg(n, f2, k)(g2, src_p, dst_p, z2)
    g3 = _tc(_m2_body, jax.ShapeDtypeStruct((n, fd), jnp.float32),
             p2, g2, dinv, W3p, b2.reshape(1, -1))
    p3 = _make_sc_agg(n, fd, k)(g3, src_p, dst_p, zd)
    out = _tc(_m4_body, jax.ShapeDtypeStruct((n, 2), jnp.float32),
              p3, g3, dinv, b3.reshape(1, -1))
    return out


# speedup vs baseline: 24.9865x; 24.9865x over previous
import functools
import math

import jax
import jax.numpy as jnp
from jax import lax
from jax.experimental import pallas as pl
from jax.experimental.pallas import tpu as pltpu
from jax.experimental.pallas import tpu_sc as plsc

N_CORES = 2
N_SUBCORES = 16
N_TILES = N_CORES * N_SUBCORES
LANES = 128

_SC_PARAMS = pltpu.CompilerParams(use_tc_tiling_on_sc=False)


@functools.lru_cache(maxsize=None)
def _make_sc_agg(n_pad, f, k):
    rows = n_pad // N_SUBCORES
    mesh = plsc.VectorSubcoreMesh(core_axis_name="c", subcore_axis_name="s")

    @functools.partial(
        pl.kernel,
        out_type=jax.ShapeDtypeStruct((N_CORES, n_pad, f), jnp.float32),
        mesh=mesh,
        scratch_types=[
            pltpu.VMEM((k, LANES), jnp.int32),
            pltpu.VMEM((k, LANES), jnp.int32),
            pltpu.VMEM((LANES, f), jnp.float32),
            pltpu.VMEM_SHARED((n_pad, f), jnp.float32),
        ],
        compiler_params=_SC_PARAMS,
    )
    def agg(g_hbm, src_hbm, dst_hbm, z_hbm, out_hbm, src_v, dst_v, buf, acc):
        c = lax.axis_index("c")
        s = lax.axis_index("s")
        wid = s * N_CORES + c
        pltpu.sync_copy(z_hbm, acc.at[pl.ds(s * rows, rows)])
        pltpu.sync_copy(src_hbm.at[wid], src_v)
        pltpu.sync_copy(dst_hbm.at[wid], dst_v)
        plsc.subcore_barrier()

        @pl.loop(0, k)
        def _(j):
            pltpu.sync_copy(g_hbm.at[src_v.at[j]], buf)
            pltpu.sync_copy(buf, acc.at[dst_v.at[j]], add=True)

        plsc.subcore_barrier()
        pltpu.sync_copy(
            acc.at[pl.ds(s * rows, rows)],
            out_hbm.at[c].at[pl.ds(s * rows, rows)],
        )

    return agg


@functools.lru_cache(maxsize=None)
def _make_sc_degree(n_pad, f, k):
    rows = n_pad // N_SUBCORES
    mesh = plsc.VectorSubcoreMesh(core_axis_name="c", subcore_axis_name="s")

    @functools.partial(
        pl.kernel,
        out_type=jax.ShapeDtypeStruct((N_CORES, n_pad, f), jnp.float32),
        mesh=mesh,
        scratch_types=[
            pltpu.VMEM((k, LANES), jnp.int32),
            pltpu.VMEM((LANES, f), jnp.float32),
            pltpu.VMEM_SHARED((n_pad, f), jnp.float32),
        ],
        compiler_params=_SC_PARAMS,
    )
    def degk(dst_hbm, ones_hbm, z_hbm, out_hbm, dst_v, ones_v, acc):
        c = lax.axis_index("c")
        s = lax.axis_index("s")
        wid = s * N_CORES + c
        pltpu.sync_copy(z_hbm, acc.at[pl.ds(s * rows, rows)])
        pltpu.sync_copy(dst_hbm.at[wid], dst_v)
        pltpu.sync_copy(ones_hbm, ones_v)
        plsc.subcore_barrier()

        @pl.loop(0, k)
        def _(j):
            pltpu.sync_copy(ones_v, acc.at[dst_v.at[j]], add=True)

        plsc.subcore_barrier()
        pltpu.sync_copy(
            acc.at[pl.ds(s * rows, rows)],
            out_hbm.at[c].at[pl.ds(s * rows, rows)],
        )

    return degk


def _dot(a, b):
    return jax.lax.dot_general(
        a, b, (((1,), (0,)), ((), ())),
        precision=jax.lax.Precision.HIGHEST,
        preferred_element_type=jnp.float32,
    )


def _m1_body(pdeg_ref, x_ref, w_ref, dinv_ref, g_ref):
    nn = dinv_ref.shape[0]
    deg = pdeg_ref[0, :nn, :1] + pdeg_ref[1, :nn, :1] + 1.0
    dinv = jax.lax.rsqrt(jnp.maximum(deg, 1e-12))
    dinv_ref[...] = dinv
    g_ref[...] = _dot(x_ref[...], w_ref[...]) * dinv


def _m2_body(p_ref, g_ref, dinv_ref, w_ref, b_ref, out_ref):
    dinv = dinv_ref[...]
    nn = g_ref.shape[0]
    p = p_ref[0, :nn, :] + p_ref[1, :nn, :]
    h = jnp.maximum(dinv * (p + g_ref[...]) + b_ref[...], 0.0)
    out_ref[...] = _dot(h, w_ref[...]) * dinv


def _m4_body(p_ref, g_ref, dinv_ref, b_ref, out_ref):
    nn = g_ref.shape[0]
    p = p_ref[0, :nn, :] + p_ref[1, :nn, :]
    t = dinv_ref[...] * (p + g_ref[...])
    logits = t[:, :2] + b_ref[...]
    m = jnp.maximum(logits[:, :1], logits[:, 1:2])
    e0 = jnp.exp(logits[:, :1] - m)
    e1 = jnp.exp(logits[:, 1:2] - m)
    lse = jnp.log(e0 + e1) + m
    out_ref[...] = logits - lse


def _tc(body, out_shapes, *args):
    return pl.pallas_call(body, out_shape=out_shapes)(*args)


def kernel(x, edge_index, W1, b1, W2, b2, W3, b3):
    n, _ = x.shape
    e = edge_index.shape[1]
    k = math.ceil(e / (N_TILES * LANES))
    ep = N_TILES * LANES * k

    src = edge_index[0].astype(jnp.int32)
    dst = edge_index[1].astype(jnp.int32)
    pad = ep - e
    src_p = jnp.concatenate([src, jnp.zeros((pad,), jnp.int32)]).reshape(
        N_TILES, k, LANES)
    dst_p = jnp.concatenate([dst, jnp.full((pad,), n, jnp.int32)]).reshape(
        N_TILES, k, LANES)

    f1 = W1.shape[1]
    f2 = W2.shape[1]
    fd = 16
    n_pad = -(-n // (N_SUBCORES * 8)) * (N_SUBCORES * 8)
    rows = n_pad // N_SUBCORES
    z1 = jnp.zeros((rows, f1), jnp.float32)
    z2 = jnp.zeros((rows, f2), jnp.float32)
    zd = jnp.zeros((rows, fd), jnp.float32)
    ones = jnp.ones((LANES, fd), jnp.float32)
    W3p = jnp.pad(W3, ((0, 0), (0, fd - W3.shape[1])))

    pdeg = _make_sc_degree(n_pad, fd, k)(dst_p, ones, zd)
    dinv, g1 = _tc(
        _m1_body,
        (jax.ShapeDtypeStruct((n, 1), jnp.float32),
         jax.ShapeDtypeStruct((n, f1), jnp.float32)),
        pdeg, x, W1)
    p1 = _make_sc_agg(n_pad, f1, k)(g1, src_p, dst_p, z1)
    g2 = _tc(_m2_body, jax.ShapeDtypeStruct((n, f2), jnp.float32),
             p1, g1, dinv, W2, b1.reshape(1, -1))
    p2 = _make_sc_agg(n_pad, f2, k)(g2, src_p, dst_p, z2)
    g3 = _tc(_m2_body, jax.ShapeDtypeStruct((n, fd), jnp.float32),
             p2, g2, dinv, W3p, b2.reshape(1, -1))
    p3 = _make_sc_agg(n_pad, fd, k)(g3, src_p, dst_p, zd)
    out = _tc(_m4_body, jax.ShapeDtypeStruct((n, 2), jnp.float32),
              p3, g3, dinv, b3.reshape(1, -1))
    return out
